# baseline (speedup 1.0000x reference)
import functools

import jax
import jax.numpy as jnp
from jax import lax
from jax.experimental import pallas as pl
from jax.experimental.pallas import tpu as pltpu
from jax.experimental.pallas import tpu_sc as plsc

_N = 100000
_D = 128
_R = 14
_E = 40000
_L = 4

_BN = 128
_NB = 784
_NP = _NB * _BN
_W = 128
_NWT = 137
_EPS = 32 * _NWT * _W
_ZROW = _NP * 15 - 16


def _h_mm_body(x_ref, w_ref, o_ref):
    o_ref[...] = jnp.dot(x_ref[...], w_ref[...],
                         preferred_element_type=jnp.float32)


def _h_matmul(x, wcat):
    bn = 512
    return pl.pallas_call(
        _h_mm_body,
        grid=(_NP // bn,),
        in_specs=[pl.BlockSpec((bn, _D), lambda i: (i, 0)),
                  pl.BlockSpec((_D, 15 * _D), lambda i: (0, 0))],
        out_specs=pl.BlockSpec((bn, 15 * _D), lambda i: (i, 0)),
        out_shape=jax.ShapeDtypeStruct((_NP, 15 * _D), jnp.float32),
    )(x, wcat)


def _gather_body(h15_h, src_h, m_h, idxw, rows, sem):
    t = lax.axis_index("s") * 2 + lax.axis_index("c")
    base = t * (_NWT * _W)

    def body(wi, carry):
        off = base + wi * _W
        pltpu.sync_copy(src_h.at[pl.ds(off, _W)], idxw)
        pltpu.async_copy(h15_h.at[idxw], rows, sem).wait()
        pltpu.sync_copy(rows, m_h.at[pl.ds(off, _W), pl.ds(0, _D)])
        return carry

    lax.fori_loop(0, _NWT, body, 0)


def _sc_gather(hmat, src_sorted):
    h15 = hmat.reshape(_NP * 15, _D)
    f = pl.kernel(
        _gather_body,
        mesh=plsc.VectorSubcoreMesh(core_axis_name="c",
                                    subcore_axis_name="s"),
        out_type=jax.ShapeDtypeStruct((_EPS, _D), jnp.float32),
        scratch_types=[
            pltpu.VMEM((_W,), jnp.int32),
            pltpu.VMEM((_W, _D), jnp.float32),
            pltpu.SemaphoreType.DMA,
        ],
    )
    return f(h15, src_sorted)


def _assemble_body(fw_ref, nw_ref, hmat_ref, res_ref, w2_ref,
                   g1w_ref, g1b_ref, g2w_ref, g2b_ref, u2d_ref, m_ref,
                   o_ref, mwin, uwin, sem_m, sem_u):
    b = pl.program_id(0)
    fw = fw_ref[b]
    nw = nw_ref[b]
    rowids = b * _BN + lax.broadcasted_iota(jnp.int32, (_BN, 1), 0)
    acc0 = hmat_ref[...]

    def start(wi):
        buf = (wi % 2) * _W
        r = fw + wi
        pltpu.make_async_copy(u2d_ref.at[pl.ds(r, 1), :],
                              uwin.at[pl.ds(wi % 2, 1), :], sem_u).start()
        pltpu.make_async_copy(m_ref.at[pl.ds(r * _W, _W), :],
                              mwin.at[pl.ds(buf, _W), :], sem_m).start()

    @pl.when(nw > 0)
    def _():
        start(0)

    def body(wi, acc):
        buf = (wi % 2) * _W
        pltpu.make_async_copy(u2d_ref.at[pl.ds(fw + wi, 1), :],
                              uwin.at[pl.ds(wi % 2, 1), :], sem_u).wait()
        pltpu.make_async_copy(m_ref.at[pl.ds((fw + wi) * _W, _W), :],
                              mwin.at[pl.ds(buf, _W), :], sem_m).wait()

        @pl.when(wi + 1 < nw)
        def _():
            start(wi + 1)

        p = (uwin[pl.ds(wi % 2, 1), :] == rowids).astype(jnp.float32)
        return acc + jnp.dot(p, mwin[pl.ds(buf, _W), :],
                             preferred_element_type=jnp.float32)

    t = lax.fori_loop(0, nw, body, acc0)
    mu = jnp.mean(t, axis=-1, keepdims=True)
    var = jnp.mean((t - mu) ** 2, axis=-1, keepdims=True)
    x = (t - mu) * lax.rsqrt(var + 1e-5) * g1w_ref[...] + g1b_ref[...]
    x = jnp.maximum(x, 0.0)
    y = jnp.dot(x, w2_ref[...], preferred_element_type=jnp.float32)
    mu2 = jnp.mean(y, axis=-1, keepdims=True)
    var2 = jnp.mean((y - mu2) ** 2, axis=-1, keepdims=True)
    y = (y - mu2) * lax.rsqrt(var2 + 1e-5) * g2w_ref[...] + g2b_ref[...]
    o_ref[...] = jnp.maximum(y + res_ref[...], 0.0)


def _assemble(first_win, nwin, hmat, res, w2t, g1w, g1b, g2w, g2b,
              u2d, m):
    vec = pl.BlockSpec((1, _D), lambda i, *_: (0, 0))
    grid_spec = pltpu.PrefetchScalarGridSpec(
        num_scalar_prefetch=2,
        grid=(_NB,),
        in_specs=[
            pl.BlockSpec((_BN, _D), lambda i, *_: (i, 0)),
            pl.BlockSpec((_BN, _D), lambda i, *_: (i, 0)),
            pl.BlockSpec((_D, _D), lambda i, *_: (0, 0)),
            vec, vec, vec, vec,
            pl.BlockSpec(memory_space=pl.ANY),
            pl.BlockSpec(memory_space=pl.ANY),
        ],
        out_specs=pl.BlockSpec((_BN, _D), lambda i, *_: (i, 0)),
        scratch_shapes=[
            pltpu.VMEM((2 * _W, _D), jnp.float32),
            pltpu.VMEM((2, _W), jnp.int32),
            pltpu.SemaphoreType.DMA,
            pltpu.SemaphoreType.DMA,
        ],
    )
    return pl.pallas_call(
        _assemble_body,
        grid_spec=grid_spec,
        out_shape=jax.ShapeDtypeStruct((_NP, _D), jnp.float32),
    )(first_win, nwin, hmat, res, w2t, g1w, g1b, g2w, g2b, u2d, m)


def kernel(feat, W_ctr, W_rel, gn1_w, gn1_b, W_ctr2, gn2_w, gn2_b,
           u_idx, v_idx):
    feat_p = jnp.pad(feat, ((0, _NP - _N), (0, 0)))
    wk_all = jnp.concatenate([W_ctr[:, None], W_rel], axis=1)
    wcat = wk_all.transpose(0, 3, 1, 2).reshape(_L, _D, 15 * _D)
    w2t = W_ctr2.transpose(0, 2, 1)

    u_flat = u_idx.reshape(-1)
    src_all = (v_idx * 15 + (jnp.arange(_R, dtype=jnp.int32) + 1)[:, None]
               ).reshape(-1)
    keys = jnp.sort((u_flat // _BN) * (1 << 20)
                    + jnp.arange(_R * _E, dtype=jnp.int32))
    pos = keys & ((1 << 20) - 1)
    u_sorted = jnp.concatenate(
        [u_flat[pos], jnp.full((_EPS - _R * _E,), _N, jnp.int32)])
    src_sorted = jnp.concatenate(
        [src_all[pos],
         _ZROW + (jnp.arange(_EPS - _R * _E, dtype=jnp.int32) % 16)])
    u2d = u_sorted.reshape(_EPS // _W, _W)
    bounds = jnp.searchsorted(
        u_sorted, jnp.arange(_NB + 1, dtype=jnp.int32) * _BN).astype(
            jnp.int32)
    start, end = bounds[:-1], bounds[1:]
    first_win = start // _W
    nwin = jnp.where(end > start, (end + _W - 1) // _W - first_win, 0)

    res = feat_p
    x = feat_p
    for i in range(_L):
        h = _h_matmul(x, wcat[i])
        m = _sc_gather(h, src_sorted)
        x = _assemble(first_win, nwin, h, res,
                      w2t[i],
                      gn1_w[i][None, :], gn1_b[i][None, :],
                      gn2_w[i][None, :], gn2_b[i][None, :],
                      u2d, m)
        res = x
    return x[:_N]


# speedup vs baseline: 2.2004x; 2.2004x over previous
import functools

import jax
import jax.numpy as jnp
from jax import lax
from jax.experimental import pallas as pl
from jax.experimental.pallas import tpu as pltpu
from jax.experimental.pallas import tpu_sc as plsc

_N = 100000
_D = 128
_R = 14
_E = 40000
_L = 4

_BN = 128
_NB = 784
_NP = _NB * _BN
_W = 128
_NWT = 138
_KW = 8
_EPS = 32 * _NWT * _W
_ZROW = _NP * 15 - 16


def _h_mm_body(x_ref, w_ref, o_ref):
    o_ref[...] = jnp.dot(x_ref[...], w_ref[...],
                         preferred_element_type=jnp.float32)


def _h_matmul(x, wcat):
    bn = 512
    return pl.pallas_call(
        _h_mm_body,
        grid=(_NP // bn,),
        in_specs=[pl.BlockSpec((bn, _D), lambda i: (i, 0)),
                  pl.BlockSpec((_D, 15 * _D), lambda i: (0, 0))],
        out_specs=pl.BlockSpec((bn, 15 * _D), lambda i: (i, 0)),
        out_shape=jax.ShapeDtypeStruct((_NP, 15 * _D), jnp.float32),
    )(x, wcat)


def _gather_body(h15_h, src_h, m_h, idxw, rows, gsem, wsem):
    t = lax.axis_index("s") * 2 + lax.axis_index("c")
    base = t * (_NWT * _W)

    def start_g(wi):
        pltpu.sync_copy(src_h.at[pl.ds(base + wi * _W, _W)], idxw)
        pltpu.make_async_copy(
            h15_h.at[idxw],
            rows.at[pl.ds((wi % 2) * _W, _W), :], gsem).start()

    start_g(0)

    def body(wi, carry):
        buf = (wi % 2) * _W
        pltpu.make_async_copy(
            h15_h.at[idxw],
            rows.at[pl.ds(buf, _W), :], gsem).wait()
        pltpu.make_async_copy(
            rows.at[pl.ds(buf, _W), :],
            m_h.at[pl.ds(base + wi * _W, _W), pl.ds(0, _D)], wsem).start()

        @pl.when(wi >= 1)
        def _():
            pltpu.make_async_copy(
                rows.at[pl.ds(((wi + 1) % 2) * _W, _W), :],
                m_h.at[pl.ds(base + (wi - 1) * _W, _W), pl.ds(0, _D)],
                wsem).wait()

        @pl.when(wi + 1 < _NWT)
        def _():
            start_g(wi + 1)
        return carry

    lax.fori_loop(0, _NWT, body, 0)
    pltpu.make_async_copy(
        rows.at[pl.ds(((_NWT - 1) % 2) * _W, _W), :],
        m_h.at[pl.ds(base + (_NWT - 1) * _W, _W), pl.ds(0, _D)],
        wsem).wait()


def _sc_gather(hmat, src_sorted):
    h15 = hmat.reshape(_NP * 15, _D)
    f = pl.kernel(
        _gather_body,
        mesh=plsc.VectorSubcoreMesh(core_axis_name="c",
                                    subcore_axis_name="s"),
        out_type=jax.ShapeDtypeStruct((_EPS, _D), jnp.float32),
        scratch_types=[
            pltpu.VMEM((_W,), jnp.int32),
            pltpu.VMEM((2 * _W, _D), jnp.float32),
            pltpu.SemaphoreType.DMA,
            pltpu.SemaphoreType.DMA,
        ],
    )
    return f(h15, src_sorted)


def _assemble_body(fw_ref, nw_ref, hmat_ref, res_ref, w2_ref,
                   g1w_ref, g1b_ref, g2w_ref, g2b_ref, u2d_ref, m_ref,
                   o_ref, mwin, uwin, sem_m, sem_u):
    b = pl.program_id(0)
    fw = fw_ref[b]
    nw = nw_ref[b]
    rowids = b * _BN + lax.broadcasted_iota(jnp.int32, (_BN, 1), 0)
    acc0 = hmat_ref[...]

    nch = (nw + _KW - 1) // _KW

    def start(ci):
        mw = fw + ci * _KW
        pltpu.make_async_copy(
            u2d_ref.at[pl.ds(mw, _KW), :],
            uwin.at[pl.ds((ci % 2) * _KW, _KW), :], sem_u).start()
        pltpu.make_async_copy(
            m_ref.at[pl.ds(mw * _W, _KW * _W), :],
            mwin.at[pl.ds((ci % 2) * _KW * _W, _KW * _W), :], sem_m).start()

    @pl.when(nch > 0)
    def _():
        start(0)

    def body(ci, acc):
        mw = fw + ci * _KW
        pltpu.make_async_copy(
            u2d_ref.at[pl.ds(mw, _KW), :],
            uwin.at[pl.ds((ci % 2) * _KW, _KW), :], sem_u).wait()
        pltpu.make_async_copy(
            m_ref.at[pl.ds(mw * _W, _KW * _W), :],
            mwin.at[pl.ds((ci % 2) * _KW * _W, _KW * _W), :], sem_m).wait()

        @pl.when(ci + 1 < nch)
        def _():
            start(ci + 1)

        for k in range(_KW):
            p = (uwin[pl.ds((ci % 2) * _KW + k, 1), :]
                 == rowids).astype(jnp.float32)
            acc = acc + jnp.dot(
                p, mwin[pl.ds(((ci % 2) * _KW + k) * _W, _W), :],
                preferred_element_type=jnp.float32)
        return acc

    t = lax.fori_loop(0, nch, body, acc0)
    mu = jnp.mean(t, axis=-1, keepdims=True)
    var = jnp.mean((t - mu) ** 2, axis=-1, keepdims=True)
    x = (t - mu) * lax.rsqrt(var + 1e-5) * g1w_ref[...] + g1b_ref[...]
    x = jnp.maximum(x, 0.0)
    y = jnp.dot(x, w2_ref[...], preferred_element_type=jnp.float32)
    mu2 = jnp.mean(y, axis=-1, keepdims=True)
    var2 = jnp.mean((y - mu2) ** 2, axis=-1, keepdims=True)
    y = (y - mu2) * lax.rsqrt(var2 + 1e-5) * g2w_ref[...] + g2b_ref[...]
    o_ref[...] = jnp.maximum(y + res_ref[...], 0.0)


def _assemble(first_win, nwin, hmat, res, w2t, g1w, g1b, g2w, g2b,
              u2d, m):
    vec = pl.BlockSpec((1, _D), lambda i, *_: (0, 0))
    grid_spec = pltpu.PrefetchScalarGridSpec(
        num_scalar_prefetch=2,
        grid=(_NB,),
        in_specs=[
            pl.BlockSpec((_BN, _D), lambda i, *_: (i, 0)),
            pl.BlockSpec((_BN, _D), lambda i, *_: (i, 0)),
            pl.BlockSpec((_D, _D), lambda i, *_: (0, 0)),
            vec, vec, vec, vec,
            pl.BlockSpec(memory_space=pl.ANY),
            pl.BlockSpec(memory_space=pl.ANY),
        ],
        out_specs=pl.BlockSpec((_BN, _D), lambda i, *_: (i, 0)),
        scratch_shapes=[
            pltpu.VMEM((2 * _KW * _W, _D), jnp.float32),
            pltpu.VMEM((2 * _KW, _W), jnp.int32),
            pltpu.SemaphoreType.DMA,
            pltpu.SemaphoreType.DMA,
        ],
    )
    return pl.pallas_call(
        _assemble_body,
        grid_spec=grid_spec,
        out_shape=jax.ShapeDtypeStruct((_NP, _D), jnp.float32),
    )(first_win, nwin, hmat, res, w2t, g1w, g1b, g2w, g2b, u2d, m)


def kernel(feat, W_ctr, W_rel, gn1_w, gn1_b, W_ctr2, gn2_w, gn2_b,
           u_idx, v_idx):
    feat_p = jnp.pad(feat, ((0, _NP - _N), (0, 0)))
    wk_all = jnp.concatenate([W_ctr[:, None], W_rel], axis=1)
    wcat = wk_all.transpose(0, 3, 1, 2).reshape(_L, _D, 15 * _D)
    w2t = W_ctr2.transpose(0, 2, 1)

    u_flat = u_idx.reshape(-1)
    src_all = (v_idx * 15 + (jnp.arange(_R, dtype=jnp.int32) + 1)[:, None]
               ).reshape(-1)
    keys = jnp.sort((u_flat // _BN) * (1 << 20)
                    + jnp.arange(_R * _E, dtype=jnp.int32))
    pos = keys & ((1 << 20) - 1)
    u_sorted = jnp.concatenate(
        [u_flat[pos], jnp.full((_EPS - _R * _E,), _NP, jnp.int32)])
    src_sorted = jnp.concatenate(
        [src_all[pos],
         _ZROW + (jnp.arange(_EPS - _R * _E, dtype=jnp.int32) % 16)])
    u2d = u_sorted.reshape(_EPS // _W, _W)
    bounds = jnp.searchsorted(
        u_sorted, jnp.arange(_NB + 1, dtype=jnp.int32) * _BN).astype(
            jnp.int32)
    start, end = bounds[:-1], bounds[1:]
    first_win = start // _W
    nwin = jnp.where(end > start, (end + _W - 1) // _W - first_win, 0)

    res = feat_p
    x = feat_p
    for i in range(_L):
        h = _h_matmul(x, wcat[i])
        m = _sc_gather(h, src_sorted)
        x = _assemble(first_win, nwin, h, res,
                      w2t[i],
                      gn1_w[i][None, :], gn1_b[i][None, :],
                      gn2_w[i][None, :], gn2_b[i][None, :],
                      u2d, m)
        res = x
    return x[:_N]
